# baseline (speedup 1.0000x reference)
import functools

import jax
import jax.numpy as jnp
from jax import lax
from jax.experimental import pallas as pl
from jax.experimental.pallas import tpu as pltpu
from jax.experimental.pallas import tpu_sc as plsc

NSPLITS = 8
ALIGN = 128
LANES = 16
CB = 16
GAP_WPG = 4
NW = 32


def _select(e, values):
    acc = values[0]
    for i in range(1, len(values)):
        acc = jnp.where(e == i, values[i], acc)
    return acc


@functools.cache
def _make_dispatch(in_len, out_len, d):
    rows_per_w = in_len // NW
    nchunks = rows_per_w // CB
    assert rows_per_w % CB == 0
    mesh = plsc.VectorSubcoreMesh(core_axis_name="c", subcore_axis_name="s")

    @functools.partial(
        pl.kernel,
        out_type=jax.ShapeDtypeStruct((out_len, d), jnp.float32),
        mesh=mesh,
        scratch_types=[
            pltpu.VMEM((LANES,), jnp.int32),
            pltpu.VMEM((CB, d), jnp.float32),
            pltpu.VMEM((CB, d), jnp.float32),
            pltpu.VMEM((CB, d), jnp.float32),
            pltpu.SemaphoreType.DMA,
            pltpu.SemaphoreType.DMA,
            pltpu.SemaphoreType.DMA,
        ],
    )
    def dispatch(inp_h, out_h, splits_h, res_h, splits_v, buf0, buf1, zbuf,
                 s0, s1, s2):
        wid = lax.axis_index("s") * 2 + lax.axis_index("c")
        pltpu.sync_copy(splits_h, splits_v)
        sv = splits_v[...]

        ends, dends, onexts, shifts = [], [], [], []
        end_acc = jnp.int32(0)
        off_acc = jnp.int32(0)
        for e in range(NSPLITS):
            s = sv[e]
            shifts.append(off_acc - end_acc)
            end_acc = end_acc + s
            ends.append(end_acc)
            dends.append(off_acc + s)
            off_acc = off_acc + ((s + (ALIGN - 1)) & jnp.int32(-ALIGN))
            onexts.append(off_acc)
        onexts[NSPLITS - 1] = jnp.int32(out_len)

        lane = lax.broadcasted_iota(jnp.int32, (LANES,), 0)
        bufs = (buf0, buf1)
        sems = (s0, s1)

        def chunk_wait(p):
            pltpu.make_async_copy(
                bufs[p], res_h.at[pl.ds(0, CB)], sems[p]).wait()

        wlo = wid * rows_per_w

        def chunk(k, carry):
            base = wlo + k * CB
            r = base + lane
            sh = jnp.full((LANES,), shifts[0], jnp.int32)
            for e in range(1, NSPLITS):
                sh = jnp.where(r >= ends[e - 1], shifts[e], sh)
            idx = r + sh
            dst0 = base + sh[0]
            linear = jnp.logical_and(sh[0] == sh[LANES - 1],
                                     (dst0 & 7) == 0)
            for par in range(2):
                @pl.when((k & 1) == par)
                def _go(par=par):
                    @pl.when(k >= 2)
                    def _drain():
                        chunk_wait(par)
                    pltpu.sync_copy(
                        inp_h.at[pl.ds(pl.multiple_of(base, CB), CB)],
                        bufs[par])

                    @pl.when(linear)
                    def _lin():
                        pltpu.async_copy(
                            bufs[par],
                            res_h.at[pl.ds(pl.multiple_of(dst0, 8), CB)],
                            sems[par])

                    @pl.when(jnp.logical_not(linear))
                    def _ind():
                        pltpu.async_copy(bufs[par], res_h.at[idx], sems[par])
            return carry

        lax.fori_loop(0, nchunks, chunk, 0)
        chunk_wait(0)
        chunk_wait(1)

        gpre = [jnp.int32(0)]
        for e in range(NSPLITS):
            gpre.append(gpre[-1] + jnp.maximum(onexts[e] - dends[e], 0))
        gtot = gpre[-1]
        share = (gtot + NW - 1) // NW
        zl = wid * share
        zh = jnp.minimum(zl + share, gtot)

        @pl.when(zh > zl)
        def _zload():
            pltpu.sync_copy(out_h.at[pl.ds(0, CB)], zbuf)

        zc = jnp.int32(0)
        for e in range(NSPLITS):
            lo = jnp.maximum(zl, gpre[e])
            hi = jnp.minimum(zh, gpre[e + 1])
            cnt = jnp.maximum(hi - lo, 0)
            dstbase = dends[e] + (lo - gpre[e])
            nzc = (cnt + CB - 1) // CB

            def zissue(i, c, dstbase=dstbase, cnt=cnt):
                idxz = jnp.minimum(dstbase + i * CB + lane,
                                   dstbase + cnt - 1)
                pltpu.async_copy(zbuf, res_h.at[idxz], s2)
                return c

            lax.fori_loop(0, nzc, zissue, 0)
            zc = zc + nzc

        def zdrain(i, c):
            pltpu.make_async_copy(zbuf, res_h.at[pl.ds(0, CB)], s2).wait()
            return c

        lax.fori_loop(0, zc, zdrain, 0)

    return dispatch


def kernel(inp, out, in_splits, out_splits_offsets):
    splits16 = jnp.zeros((LANES,), jnp.int32).at[:NSPLITS].set(
        in_splits.astype(jnp.int32))
    f = _make_dispatch(inp.shape[0], out.shape[0], inp.shape[1])
    return f(inp, out, splits16)


# speedup vs baseline: 1.0060x; 1.0060x over previous
import functools

import jax
import jax.numpy as jnp
from jax import lax
from jax.experimental import pallas as pl
from jax.experimental.pallas import tpu as pltpu
from jax.experimental.pallas import tpu_sc as plsc

NSPLITS = 8
ALIGN = 128
LANES = 16
CB = 16
NBUF = 3
NW = 32


@functools.cache
def _make_dispatch(in_len, out_len, d):
    rows_per_w = in_len // NW
    nchunks = rows_per_w // CB
    assert rows_per_w % CB == 0 and nchunks >= NBUF
    mesh = plsc.VectorSubcoreMesh(core_axis_name="c", subcore_axis_name="s")

    @functools.partial(
        pl.kernel,
        out_type=jax.ShapeDtypeStruct((out_len, d), jnp.float32),
        mesh=mesh,
        scratch_types=[
            pltpu.VMEM((LANES,), jnp.int32),
            pltpu.VMEM((CB, d), jnp.float32),
            pltpu.VMEM((CB, d), jnp.float32),
            pltpu.VMEM((CB, d), jnp.float32),
            pltpu.SemaphoreType.DMA,
            pltpu.SemaphoreType.DMA,
            pltpu.SemaphoreType.DMA,
            pltpu.SemaphoreType.DMA,
            pltpu.SemaphoreType.DMA,
            pltpu.SemaphoreType.DMA,
        ],
    )
    def dispatch(inp_h, out_h, splits_h, res_h, splits_v, b0, b1, b2,
                 g0, g1, g2, s0, s1, s2):
        wid = lax.axis_index("s") * 2 + lax.axis_index("c")
        pltpu.sync_copy(splits_h, splits_v)
        sv = splits_v[...]

        ends, dends, onexts, shifts = [], [], [], []
        end_acc = jnp.int32(0)
        off_acc = jnp.int32(0)
        for e in range(NSPLITS):
            s = sv[e]
            shifts.append(off_acc - end_acc)
            end_acc = end_acc + s
            ends.append(end_acc)
            dends.append(off_acc + s)
            off_acc = off_acc + ((s + (ALIGN - 1)) & jnp.int32(-ALIGN))
            onexts.append(off_acc)
        onexts[NSPLITS - 1] = jnp.int32(out_len)

        lane = lax.broadcasted_iota(jnp.int32, (LANES,), 0)
        bufs = (b0, b1, b2)
        gsems = (g0, g1, g2)
        ssems = (s0, s1, s2)
        wlo = wid * rows_per_w

        def gwait(p):
            pltpu.make_async_copy(
                inp_h.at[pl.ds(0, CB)], bufs[p], gsems[p]).wait()

        def swait(p):
            pltpu.make_async_copy(
                bufs[p], res_h.at[pl.ds(0, CB)], ssems[p]).wait()

        def gissue(k, p):
            base = pl.multiple_of(wlo + k * CB, CB)
            pltpu.async_copy(inp_h.at[pl.ds(base, CB)], bufs[p], gsems[p])

        gissue(0, 0)
        gissue(1, 1)

        def chunk(k, carry):
            r = wlo + k * CB + lane
            sh = jnp.full((LANES,), shifts[0], jnp.int32)
            for e in range(1, NSPLITS):
                sh = jnp.where(r >= ends[e - 1], shifts[e], sh)
            idx = r + sh
            for p in range(NBUF):
                @pl.when(k % NBUF == p)
                def _go(p=p):
                    gwait(p)
                    pltpu.async_copy(bufs[p], res_h.at[idx], ssems[p])

            @pl.when(k + 2 < nchunks)
            def _prefetch():
                for p in range(NBUF):
                    @pl.when((k + 2) % NBUF == p)
                    def _gi(p=p):
                        @pl.when(k >= 1)
                        def _sw():
                            swait(p)
                        base2 = pl.multiple_of(wlo + (k + 2) * CB, CB)
                        pltpu.async_copy(
                            inp_h.at[pl.ds(base2, CB)], bufs[p], gsems[p])
            return carry

        lax.fori_loop(0, nchunks, chunk, 0)
        swait((nchunks - 3) % NBUF)
        swait((nchunks - 2) % NBUF)
        swait((nchunks - 1) % NBUF)

        gpre = [jnp.int32(0)]
        for e in range(NSPLITS):
            gpre.append(gpre[-1] + jnp.maximum(onexts[e] - dends[e], 0))
        gtot = gpre[-1]
        share = (gtot + NW - 1) // NW
        zl = wid * share
        zh = jnp.minimum(zl + share, gtot)

        @pl.when(zh > zl)
        def _zload():
            pltpu.sync_copy(out_h.at[pl.ds(0, CB)], b0)

        zc = jnp.int32(0)
        for e in range(NSPLITS):
            lo = jnp.maximum(zl, gpre[e])
            hi = jnp.minimum(zh, gpre[e + 1])
            cnt = jnp.maximum(hi - lo, 0)
            dstbase = dends[e] + (lo - gpre[e])
            nzc = (cnt + CB - 1) // CB

            def zissue(i, c, dstbase=dstbase, cnt=cnt):
                idxz = jnp.minimum(dstbase + i * CB + lane,
                                   dstbase + cnt - 1)
                pltpu.async_copy(b0, res_h.at[idxz], s2)
                return c

            lax.fori_loop(0, nzc, zissue, 0)
            zc = zc + nzc

        def zdrain(i, c):
            pltpu.make_async_copy(b0, res_h.at[pl.ds(0, CB)], s2).wait()
            return c

        lax.fori_loop(0, zc, zdrain, 0)

    return dispatch


def kernel(inp, out, in_splits, out_splits_offsets):
    splits16 = jnp.zeros((LANES,), jnp.int32).at[:NSPLITS].set(
        in_splits.astype(jnp.int32))
    f = _make_dispatch(inp.shape[0], out.shape[0], inp.shape[1])
    return f(inp, out, splits16)
